# baseline (speedup 1.0000x reference)
import jax
import jax.numpy as jnp
from jax import lax
from jax.experimental import pallas as pl
from jax.experimental.pallas import tpu as pltpu
from jax.experimental.pallas import tpu_sc as plsc

D = 64
SCALE = 8.0
LANES = 16
IDXROW = 128
CHUNK = 256
NBUF = 2


def kernel(x, lut):
    b0, hist = x.shape
    nw = 32
    bw = b0 // nw
    k = CHUNK // IDXROW
    cw = bw // CHUNK
    nchunks = hist * cw
    xt3 = jnp.transpose(x).reshape(hist, b0 // IDXROW, IDXROW)

    mesh = plsc.VectorSubcoreMesh(core_axis_name="c", subcore_axis_name="s")

    @pl.kernel(
        mesh=mesh,
        compiler_params=pltpu.CompilerParams(
            use_tc_tiling_on_sc=False, needs_layout_passes=False
        ),
        out_type=jax.ShapeDtypeStruct((hist, D, b0), jnp.float32),
        scratch_types=[
            [pltpu.VMEM((k, IDXROW), jnp.int32) for _ in range(NBUF)],
            [pltpu.VMEM((CHUNK, D), jnp.float32) for _ in range(NBUF)],
            [pltpu.VMEM((D, CHUNK), jnp.float32) for _ in range(NBUF)],
            [pltpu.SemaphoreType.DMA for _ in range(NBUF)],
            [pltpu.SemaphoreType.DMA for _ in range(NBUF)],
        ],
    )
    def emb(idx_hbm, tab_hbm, out_hbm, idx_v, rows_v, tr_v, gsem, osem):
        wid = lax.axis_index("s") * 2 + lax.axis_index("c")
        jrow0 = wid * (bw // IDXROW)
        bbase = wid * bw
        iota = lax.iota(jnp.int32, LANES)

        def stage_and_gather(c, buf):
            h = c // cw
            par = c % cw
            pltpu.sync_copy(
                idx_hbm.at[h, pl.ds(jrow0 + par * k, k)], idx_v[buf]
            )
            for j in range(k):
                pltpu.async_copy(
                    tab_hbm.at[idx_v[buf].at[j]],
                    rows_v[buf].at[pl.ds(j * IDXROW, IDXROW)],
                    gsem[buf],
                )

        def drain_rows(sem, buf):
            pltpu.make_async_copy(
                tab_hbm.at[pl.ds(0, CHUNK)], rows_v[buf], sem
            ).wait()

        def drain_out(sem, buf):
            pltpu.make_async_copy(
                out_hbm.at[0, :, pl.ds(0, CHUNK)], tr_v[buf], sem
            ).wait()

        stage_and_gather(0, 0)

        @pl.loop(0, nchunks, step=NBUF)
        def chunk_body(c0):
            for phase in range(NBUF):
                c = c0 + phase
                cur = phase
                nxt = (phase + 1) % NBUF

                @pl.when(c + 1 < nchunks)
                def _prefetch():
                    stage_and_gather(c + 1, nxt)

                drain_rows(gsem[cur], cur)

                @pl.when(c >= NBUF)
                def _wait_out():
                    drain_out(osem[cur], cur)

                @plsc.parallel_loop(0, D, unroll=2)
                def trans_d(d):
                    col = jnp.full((LANES,), 0, jnp.int32) + d
                    for bg in range(CHUNK // LANES):
                        rows16 = iota + (bg * LANES)
                        vals = plsc.load_gather(rows_v[cur], [rows16, col])
                        tr_v[cur][d, pl.ds(bg * LANES, LANES)] = vals * SCALE

                h = c // cw
                par = c % cw
                pltpu.async_copy(
                    tr_v[cur],
                    out_hbm.at[h, :, pl.ds(bbase + par * CHUNK, CHUNK)],
                    osem[cur],
                )

        for buf in range(NBUF):
            drain_out(osem[buf], buf)

    out3 = emb(xt3, lut)
    return jnp.transpose(out3, (2, 0, 1))
